# baseline (speedup 1.0000x reference)
import jax
import jax.numpy as jnp
from jax import lax
from jax.experimental import pallas as pl
from jax.experimental.pallas import tpu as pltpu
from jax.experimental.pallas import tpu_sc as plsc

NC, NS, L = 2, 16, 16
NW = NC * NS

B, S, D = 4, 2048, 4096
PW = S // NW
P = 8
NCHUNK = PW // P
R = 4
T = NCHUNK * 2 * B


def _sc_body(x_hbm, pos_hbm, idx_hbm, out_hbm, idx_v, pos_v, xio_v,
             sem_g, sem_in, sem_out):
    wid = lax.axis_index("s") * NC + lax.axis_index("c")
    base = wid * PW

    pltpu.sync_copy(idx_hbm.at[pl.ds(base, PW)], idx_v)

    def gather(c):
        return pltpu.make_async_copy(
            pos_hbm.at[idx_v.at[pl.ds(c * P, P)]],
            pos_v, sem_g)

    def split(t):
        c = t // (2 * B)
        u = t - c * (2 * B)
        rh = u // B
        b = u - rh * B
        return c, rh, b

    def in_copy(t):
        c, rh, b = split(t)
        return pltpu.make_async_copy(
            x_hbm.at[b, pl.ds(base + c * P + rh * R, R)],
            xio_v.at[lax.rem(t, 4)], sem_in)

    def out_copy(t):
        c, rh, b = split(t)
        return pltpu.make_async_copy(
            xio_v.at[lax.rem(t, 4)],
            out_hbm.at[b, pl.ds(base + c * P + rh * R, R)], sem_out)

    gather(0).start()
    in_copy(0).start()
    in_copy(1).start()

    def step(t, carry):
        c, rh, b = split(t)
        tb = lax.rem(t, 4)

        @pl.when(t - c * (2 * B) == 0)
        def _():
            gather(c).wait()

        @pl.when(t >= 2)
        def _():
            out_copy(t - 2).wait()

        @pl.when(t + 2 < T)
        def _():
            in_copy(t + 2).start()

        in_copy(t).wait()

        rb = rh * R

        @plsc.parallel_loop(0, D // L, unroll=8)
        def add_vecs(i):
            s = pl.ds(i * L, L)
            for r in range(R):
                plsc.addupdate(xio_v.at[tb, r, s], pos_v[rb + r, s])

        @pl.when(jnp.logical_and(t - c * (2 * B) == 2 * B - 1,
                                 c + 1 < NCHUNK))
        def _():
            gather(c + 1).start()

        out_copy(t).start()

        return carry

    lax.fori_loop(0, T, step, 0)

    out_copy(T - 2).wait()
    out_copy(T - 1).wait()


def kernel(x, pos_emb, block_indices):
    idx = block_indices.astype(jnp.int32)
    k = pl.kernel(
        _sc_body,
        out_type=jax.ShapeDtypeStruct((B, S, D), jnp.float32),
        mesh=plsc.VectorSubcoreMesh(
            core_axis_name="c", subcore_axis_name="s",
            num_cores=NC, num_subcores=NS),
        scratch_types=[
            pltpu.VMEM((PW,), jnp.int32),
            pltpu.VMEM((P, D), jnp.float32),
            pltpu.VMEM((4, R, D), jnp.float32),
            pltpu.SemaphoreType.DMA,
            pltpu.SemaphoreType.DMA,
            pltpu.SemaphoreType.DMA,
        ],
    )
    return k(x, pos_emb, idx)


# speedup vs baseline: 1.0442x; 1.0442x over previous
import jax
import jax.numpy as jnp
from jax import lax
from jax.experimental import pallas as pl
from jax.experimental.pallas import tpu as pltpu
from jax.experimental.pallas import tpu_sc as plsc

NC, NS, L = 2, 16, 16
NW = NC * NS

B, S, D = 4, 2048, 4096
PW = S // NW
P = 8
NCHUNK = PW // P
CH = D // 2
NP = NCHUNK * 2
T = NP * B


def _sc_body(x_hbm, pos_hbm, idx_hbm, out_hbm, idx_v, pos_v, xio_v,
             sem_g, sem_in, sem_out):
    wid = lax.axis_index("s") * NC + lax.axis_index("c")
    base = wid * PW

    pltpu.sync_copy(idx_hbm.at[pl.ds(base, PW)], idx_v)

    def gather(p):
        c = p // 2
        h = p - c * 2
        return pltpu.make_async_copy(
            pos_hbm.at[idx_v.at[pl.ds(c * P, P)], pl.ds(h * CH, CH)],
            pos_v.at[lax.rem(p, 2)], sem_g)

    def split(t):
        p = t // B
        b = t - p * B
        c = p // 2
        h = p - c * 2
        return p, b, c, h

    def in_copy(t):
        p, b, c, h = split(t)
        return pltpu.make_async_copy(
            x_hbm.at[b, pl.ds(base + c * P, P), pl.ds(h * CH, CH)],
            xio_v.at[lax.rem(t, 5)], sem_in)

    def out_copy(t):
        p, b, c, h = split(t)
        return pltpu.make_async_copy(
            xio_v.at[lax.rem(t, 5)],
            out_hbm.at[b, pl.ds(base + c * P, P), pl.ds(h * CH, CH)],
            sem_out)

    gather(0).start()
    in_copy(0).start()
    in_copy(1).start()
    in_copy(2).start()

    def step(t, carry):
        p, b, c, h = split(t)
        tb = lax.rem(t, 5)
        pb = lax.rem(p, 2)

        @pl.when(b == 0)
        def _():
            gather(p).wait()

        @pl.when(jnp.logical_and(b == 0, p + 1 < NP))
        def _():
            gather(p + 1).start()

        @pl.when(t >= 2)
        def _():
            out_copy(t - 2).wait()

        @pl.when(t + 3 < T)
        def _():
            in_copy(t + 3).start()

        in_copy(t).wait()

        @plsc.parallel_loop(0, CH // L, unroll=8)
        def add_vecs(i):
            s = pl.ds(i * L, L)
            for r in range(P):
                plsc.addupdate(xio_v.at[tb, r, s], pos_v[pb, r, s])

        out_copy(t).start()

        return carry

    lax.fori_loop(0, T, step, 0)

    out_copy(T - 2).wait()
    out_copy(T - 1).wait()


def kernel(x, pos_emb, block_indices):
    idx = block_indices.astype(jnp.int32)
    k = pl.kernel(
        _sc_body,
        out_type=jax.ShapeDtypeStruct((B, S, D), jnp.float32),
        mesh=plsc.VectorSubcoreMesh(
            core_axis_name="c", subcore_axis_name="s",
            num_cores=NC, num_subcores=NS),
        scratch_types=[
            pltpu.VMEM((PW,), jnp.int32),
            pltpu.VMEM((2, P, CH), jnp.float32),
            pltpu.VMEM((5, P, CH), jnp.float32),
            pltpu.SemaphoreType.DMA,
            pltpu.SemaphoreType.DMA,
            pltpu.SemaphoreType.DMA,
        ],
    )
    return k(x, pos_emb, idx)


# speedup vs baseline: 1.0534x; 1.0088x over previous
import jax
import jax.numpy as jnp
from jax import lax
from jax.experimental import pallas as pl
from jax.experimental.pallas import tpu as pltpu
from jax.experimental.pallas import tpu_sc as plsc

NC, NS, L = 2, 16, 16
NW = NC * NS

B, S, D = 4, 2048, 4096
PW = S // NW
P = 8
NCHUNK = PW // P
CH = D // 4
NP = NCHUNK * 4
T = NP * B


def _sc_body(x_hbm, pos_hbm, idx_hbm, out_hbm, idx_v, pos_v, xio_v,
             sem_g, sem_in, sem_out):
    wid = lax.axis_index("s") * NC + lax.axis_index("c")
    base = wid * PW

    pltpu.sync_copy(idx_hbm.at[pl.ds(base, PW)], idx_v)

    def gather(p):
        c = p // 4
        h = p - c * 4
        return pltpu.make_async_copy(
            pos_hbm.at[idx_v.at[pl.ds(c * P, P)], pl.ds(h * CH, CH)],
            pos_v.at[lax.rem(p, 2)], sem_g)

    def split(t):
        p = t // B
        b = t - p * B
        c = p // 4
        h = p - c * 4
        return p, b, c, h

    def in_copy(t):
        p, b, c, h = split(t)
        return pltpu.make_async_copy(
            x_hbm.at[b, pl.ds(base + c * P, P), pl.ds(h * CH, CH)],
            xio_v.at[lax.rem(t, 8)], sem_in)

    def out_copy(t):
        p, b, c, h = split(t)
        return pltpu.make_async_copy(
            xio_v.at[lax.rem(t, 8)],
            out_hbm.at[b, pl.ds(base + c * P, P), pl.ds(h * CH, CH)],
            sem_out)

    gather(0).start()
    in_copy(0).start()
    in_copy(1).start()
    in_copy(2).start()
    in_copy(3).start()

    def step(t, carry):
        p, b, c, h = split(t)
        tb = lax.rem(t, 8)
        pb = lax.rem(p, 2)

        @pl.when(b == 0)
        def _():
            gather(p).wait()

        @pl.when(jnp.logical_and(b == 0, p + 1 < NP))
        def _():
            gather(p + 1).start()

        @pl.when(t >= 4)
        def _():
            out_copy(t - 4).wait()

        @pl.when(t + 4 < T)
        def _():
            in_copy(t + 4).start()

        in_copy(t).wait()

        @plsc.parallel_loop(0, CH // L, unroll=8)
        def add_vecs(i):
            s = pl.ds(i * L, L)
            for r in range(P):
                plsc.addupdate(xio_v.at[tb, r, s], pos_v[pb, r, s])

        out_copy(t).start()

        return carry

    lax.fori_loop(0, T, step, 0)

    out_copy(T - 4).wait()
    out_copy(T - 3).wait()
    out_copy(T - 2).wait()
    out_copy(T - 1).wait()


def kernel(x, pos_emb, block_indices):
    idx = block_indices.astype(jnp.int32)
    k = pl.kernel(
        _sc_body,
        out_type=jax.ShapeDtypeStruct((B, S, D), jnp.float32),
        mesh=plsc.VectorSubcoreMesh(
            core_axis_name="c", subcore_axis_name="s",
            num_cores=NC, num_subcores=NS),
        scratch_types=[
            pltpu.VMEM((PW,), jnp.int32),
            pltpu.VMEM((2, P, CH), jnp.float32),
            pltpu.VMEM((8, P, CH), jnp.float32),
            pltpu.SemaphoreType.DMA,
            pltpu.SemaphoreType.DMA,
            pltpu.SemaphoreType.DMA,
        ],
    )
    return k(x, pos_emb, idx)
